# Initial kernel scaffold; baseline (speedup 1.0000x reference)
import jax
import jax.numpy as jnp
from jax.experimental import pallas as pl


def kernel(inputs, embedding_weight):
    raise NotImplementedError("write your pallas kernel here")


# speedup vs baseline: 1.1425x; 1.1425x over previous
import jax
import jax.numpy as jnp
from jax import lax
from jax.experimental import pallas as pl
from jax.experimental.pallas import tpu as pltpu
from jax.experimental.pallas import tpu_sc as plsc

_K = 8192
_D = 256
_N = 16384
_TN = 256
_KT = 2048

_NC = 2
_NS = 16
_NW = _NC * _NS
_BPW = _N // _NW
_CH = 128


def _dist_argmin_body(x_ref, w_ref, sx_ref, sw_ref, idx_ref):
    x = x_ref[...]
    sx = sx_ref[...]
    halves = []
    for h in range(2):
        best = jnp.full((_TN,), jnp.inf, jnp.float32)
        bidx = jnp.zeros((_TN,), jnp.int32)
        for c in range(_K // _KT // 2):
            kt = h * (_K // _KT // 2) + c
            w = w_ref[kt * _KT:(kt + 1) * _KT, :]
            mm = lax.dot_general(
                x, w, (((1,), (1,)), ((), ())),
                preferred_element_type=jnp.float32)
            d = (sx - mm) + sw_ref[:, kt * _KT:(kt + 1) * _KT]
            cmin = jnp.min(d, axis=1)
            iota = lax.broadcasted_iota(jnp.int32, (_TN, _KT), 1) + kt * _KT
            cidx = jnp.min(jnp.where(d == cmin[:, None], iota, _K), axis=1)
            upd = cmin < best
            bidx = jnp.where(upd, cidx, bidx)
            best = jnp.where(upd, cmin, best)
        halves.append((best, bidx))
    (b0, i0), (b1, i1) = halves
    state = b0.astype(jnp.bfloat16).astype(jnp.float32)
    use_hi = b1 < state
    idx_ref[...] = jnp.where(use_hi, i1, i0)[:, None]


def _sc_gather_body(w_hbm, idx_hbm, out_hbm, idx_v, rows_v, sem):
    wid = lax.axis_index("s") * _NC + lax.axis_index("c")
    base = wid * _BPW
    for c in range(_BPW // _CH):
        r0 = base + c * _CH
        pltpu.sync_copy(idx_hbm.at[pl.ds(r0, _CH)], idx_v)
        pltpu.async_copy(w_hbm.at[idx_v], rows_v, sem).wait()
        pltpu.sync_copy(rows_v, out_hbm.at[pl.ds(r0, _CH)])


def _sc_gather(embedding_weight, idx_flat):
    return pl.kernel(
        _sc_gather_body,
        out_type=jax.ShapeDtypeStruct((_N, _D), jnp.float32),
        mesh=plsc.VectorSubcoreMesh(core_axis_name="c", subcore_axis_name="s"),
        scratch_types=[
            pltpu.VMEM((_CH,), jnp.int32),
            pltpu.VMEM((_CH, _D), jnp.float32),
            pltpu.SemaphoreType.DMA,
        ],
    )(embedding_weight, idx_flat)


def _st_loss_body(x_ref, q_ref, st_ref, ps_ref):
    x = x_ref[...]
    q = q_ref[...]
    d = q - x
    st_ref[...] = x + d
    ps_ref[...] = jnp.sum(d * d).reshape(1, 1, 1)


def _loss_combine_body(ps_ref, out_ref):
    m = jnp.sum(ps_ref[...]) * (1.0 / 4194304.0)
    out_ref[...] = (0.25 * m + m).reshape(1, 1)


def kernel(inputs, embedding_weight):
    B, C, H, W = inputs.shape
    flat_x = jnp.transpose(inputs, (0, 2, 3, 1)).reshape(_N, _D)
    x_b = (2.0 * flat_x).astype(jnp.bfloat16)
    w_b = embedding_weight.astype(jnp.bfloat16)
    x_iso = jax.lax.optimization_barrier(inputs)
    sx = jnp.sum(jnp.transpose(x_iso, (0, 2, 3, 1)).reshape(_N, _D) ** 2,
                 axis=1, keepdims=True)
    sw = jnp.sum(embedding_weight ** 2, axis=1).reshape(1, _K)

    idx2d = pl.pallas_call(
        _dist_argmin_body,
        grid=(_N // _TN,),
        in_specs=[
            pl.BlockSpec((_TN, _D), lambda i: (i, 0)),
            pl.BlockSpec((_K, _D), lambda i: (0, 0)),
            pl.BlockSpec((_TN, 1), lambda i: (i, 0)),
            pl.BlockSpec((1, _K), lambda i: (0, 0)),
        ],
        out_specs=pl.BlockSpec((_TN, 1), lambda i: (i, 0)),
        out_shape=jax.ShapeDtypeStruct((_N, 1), jnp.int32),
    )(x_b, w_b, sx, sw)
    idx_flat = idx2d.reshape(_N)

    q_flat = _sc_gather(embedding_weight, idx_flat)

    st_flat, ps = pl.pallas_call(
        _st_loss_body,
        grid=(16,),
        in_specs=[
            pl.BlockSpec((_N // 16, _D), lambda i: (i, 0)),
            pl.BlockSpec((_N // 16, _D), lambda i: (i, 0)),
        ],
        out_specs=[
            pl.BlockSpec((_N // 16, _D), lambda i: (i, 0)),
            pl.BlockSpec((1, 1, 1), lambda i: (i, 0, 0)),
        ],
        out_shape=[
            jax.ShapeDtypeStruct((_N, _D), jnp.float32),
            jax.ShapeDtypeStruct((16, 1, 1), jnp.float32),
        ],
    )(flat_x, q_flat)

    loss = pl.pallas_call(
        _loss_combine_body,
        in_specs=[pl.BlockSpec((16, 1, 1), lambda: (0, 0, 0))],
        out_specs=pl.BlockSpec((1, 1), lambda: (0, 0)),
        out_shape=jax.ShapeDtypeStruct((1, 1), jnp.float32),
    )(ps)[0, 0]

    quantized_st = jnp.transpose(st_flat.reshape(B, H, W, C), (0, 3, 1, 2))
    return quantized_st, loss, idx_flat.reshape(B, H, W)


# speedup vs baseline: 1.2260x; 1.0731x over previous
import jax
import jax.numpy as jnp
from jax import lax
from jax.experimental import pallas as pl
from jax.experimental.pallas import tpu as pltpu
from jax.experimental.pallas import tpu_sc as plsc

_K = 8192
_D = 256
_N = 16384
_TN = 512
_KT = 2048

_NC = 2
_NS = 16
_NW = _NC * _NS
_BPW = _N // _NW
_CH = 128


def _dist_argmin_body(x_ref, w_ref, sx_ref, sw_ref, idx_ref):
    x = x_ref[...]
    sx = sx_ref[...]
    halves = []
    for h in range(2):
        best = jnp.full((_TN,), jnp.inf, jnp.float32)
        bidx = jnp.zeros((_TN,), jnp.int32)
        for c in range(_K // _KT // 2):
            kt = h * (_K // _KT // 2) + c
            w = w_ref[kt * _KT:(kt + 1) * _KT, :]
            mm = lax.dot_general(
                x, w, (((1,), (1,)), ((), ())),
                preferred_element_type=jnp.float32)
            d = (sx - mm) + sw_ref[:, kt * _KT:(kt + 1) * _KT]
            cmin = jnp.min(d, axis=1)
            iota = lax.broadcasted_iota(jnp.int32, (_TN, _KT), 1) + kt * _KT
            cidx = jnp.min(jnp.where(d == cmin[:, None], iota, _K), axis=1)
            upd = cmin < best
            bidx = jnp.where(upd, cidx, bidx)
            best = jnp.where(upd, cmin, best)
        halves.append((best, bidx))
    (b0, i0), (b1, i1) = halves
    state = b0.astype(jnp.bfloat16).astype(jnp.float32)
    use_hi = b1 < state
    idx_ref[...] = jnp.where(use_hi, i1, i0)[:, None]


def _sc_gather_body(w_hbm, idx_hbm, out_hbm, idx_v, rows_v, sem):
    wid = lax.axis_index("s") * _NC + lax.axis_index("c")
    base = wid * _BPW
    for c in range(_BPW // _CH):
        r0 = base + c * _CH
        pltpu.sync_copy(idx_hbm.at[pl.ds(r0, _CH)], idx_v)
        pltpu.async_copy(w_hbm.at[idx_v], rows_v, sem).wait()
        pltpu.sync_copy(rows_v, out_hbm.at[pl.ds(r0, _CH)])


def _sc_gather(embedding_weight, idx_flat):
    return pl.kernel(
        _sc_gather_body,
        out_type=jax.ShapeDtypeStruct((_N, _D), jnp.float32),
        mesh=plsc.VectorSubcoreMesh(core_axis_name="c", subcore_axis_name="s"),
        scratch_types=[
            pltpu.VMEM((_CH,), jnp.int32),
            pltpu.VMEM((_CH, _D), jnp.float32),
            pltpu.SemaphoreType.DMA,
        ],
    )(embedding_weight, idx_flat)


def _st_loss_body(x_ref, q_ref, st_ref, ps_ref):
    x = x_ref[...]
    q = q_ref[...]
    d = q - x
    st_ref[...] = x + d
    ps_ref[...] = jnp.sum(d * d).reshape(1, 1, 1)


def _loss_combine_body(ps_ref, out_ref):
    m = jnp.sum(ps_ref[...]) * (1.0 / 4194304.0)
    out_ref[...] = (0.25 * m + m).reshape(1, 1)


def kernel(inputs, embedding_weight):
    B, C, H, W = inputs.shape
    flat_x = jnp.transpose(inputs, (0, 2, 3, 1)).reshape(_N, _D)
    x_b = (2.0 * flat_x).astype(jnp.bfloat16)
    w_b = embedding_weight.astype(jnp.bfloat16)
    x_iso = jax.lax.optimization_barrier(inputs)
    sx = jnp.sum(jnp.transpose(x_iso, (0, 2, 3, 1)).reshape(_N, _D) ** 2,
                 axis=1, keepdims=True)
    sw = jnp.sum(embedding_weight ** 2, axis=1).reshape(1, _K)

    idx2d = pl.pallas_call(
        _dist_argmin_body,
        grid=(_N // _TN,),
        in_specs=[
            pl.BlockSpec((_TN, _D), lambda i: (i, 0)),
            pl.BlockSpec((_K, _D), lambda i: (0, 0)),
            pl.BlockSpec((_TN, 1), lambda i: (i, 0)),
            pl.BlockSpec((1, _K), lambda i: (0, 0)),
        ],
        out_specs=pl.BlockSpec((_TN, 1), lambda i: (i, 0)),
        out_shape=jax.ShapeDtypeStruct((_N, 1), jnp.int32),
    )(x_b, w_b, sx, sw)
    idx_flat = idx2d.reshape(_N)

    q_flat = _sc_gather(embedding_weight, idx_flat)

    st_flat, ps = pl.pallas_call(
        _st_loss_body,
        grid=(16,),
        in_specs=[
            pl.BlockSpec((_N // 16, _D), lambda i: (i, 0)),
            pl.BlockSpec((_N // 16, _D), lambda i: (i, 0)),
        ],
        out_specs=[
            pl.BlockSpec((_N // 16, _D), lambda i: (i, 0)),
            pl.BlockSpec((1, 1, 1), lambda i: (i, 0, 0)),
        ],
        out_shape=[
            jax.ShapeDtypeStruct((_N, _D), jnp.float32),
            jax.ShapeDtypeStruct((16, 1, 1), jnp.float32),
        ],
    )(flat_x, q_flat)

    loss = pl.pallas_call(
        _loss_combine_body,
        in_specs=[pl.BlockSpec((16, 1, 1), lambda: (0, 0, 0))],
        out_specs=pl.BlockSpec((1, 1), lambda: (0, 0)),
        out_shape=jax.ShapeDtypeStruct((1, 1), jnp.float32),
    )(ps)[0, 0]

    quantized_st = jnp.transpose(st_flat.reshape(B, H, W, C), (0, 3, 1, 2))
    return quantized_st, loss, idx_flat.reshape(B, H, W)


# speedup vs baseline: 1.2867x; 1.0495x over previous
import jax
import jax.numpy as jnp
from jax import lax
from jax.experimental import pallas as pl
from jax.experimental.pallas import tpu as pltpu
from jax.experimental.pallas import tpu_sc as plsc

_K = 8192
_D = 256
_N = 16384
_TN = 1024
_KT = 2048

_NC = 2
_NS = 16
_NW = _NC * _NS
_BPW = _N // _NW
_CH = 128


def _dist_argmin_body(x_ref, w_ref, sx_ref, sw_ref, idx_ref):
    x = x_ref[...]
    sx = sx_ref[...]
    halves = []
    for h in range(2):
        best = jnp.full((_TN,), jnp.inf, jnp.float32)
        bidx = jnp.zeros((_TN,), jnp.int32)
        for c in range(_K // _KT // 2):
            kt = h * (_K // _KT // 2) + c
            w = w_ref[kt * _KT:(kt + 1) * _KT, :]
            mm = lax.dot_general(
                x, w, (((1,), (1,)), ((), ())),
                preferred_element_type=jnp.float32)
            d = (sx - mm) + sw_ref[:, kt * _KT:(kt + 1) * _KT]
            cmin = jnp.min(d, axis=1)
            iota = lax.broadcasted_iota(jnp.int32, (_TN, _KT), 1) + kt * _KT
            cidx = jnp.min(jnp.where(d == cmin[:, None], iota, _K), axis=1)
            upd = cmin < best
            bidx = jnp.where(upd, cidx, bidx)
            best = jnp.where(upd, cmin, best)
        halves.append((best, bidx))
    (b0, i0), (b1, i1) = halves
    state = b0.astype(jnp.bfloat16).astype(jnp.float32)
    use_hi = b1 < state
    idx_ref[...] = jnp.where(use_hi, i1, i0)[:, None]


def _sc_gather_body(w_hbm, idx_hbm, out_hbm, idx_v, rows_v, sem):
    wid = lax.axis_index("s") * _NC + lax.axis_index("c")
    base = wid * _BPW
    for c in range(_BPW // _CH):
        r0 = base + c * _CH
        pltpu.sync_copy(idx_hbm.at[pl.ds(r0, _CH)], idx_v)
        pltpu.async_copy(w_hbm.at[idx_v], rows_v, sem).wait()
        pltpu.sync_copy(rows_v, out_hbm.at[pl.ds(r0, _CH)])


def _sc_gather(embedding_weight, idx_flat):
    return pl.kernel(
        _sc_gather_body,
        out_type=jax.ShapeDtypeStruct((_N, _D), jnp.float32),
        mesh=plsc.VectorSubcoreMesh(core_axis_name="c", subcore_axis_name="s"),
        scratch_types=[
            pltpu.VMEM((_CH,), jnp.int32),
            pltpu.VMEM((_CH, _D), jnp.float32),
            pltpu.SemaphoreType.DMA,
        ],
    )(embedding_weight, idx_flat)


def _st_loss_body(x_ref, q_ref, st_ref, ps_ref):
    x = x_ref[...]
    q = q_ref[...]
    d = q - x
    st_ref[...] = x + d
    ps_ref[...] = jnp.sum(d * d).reshape(1, 1, 1)


def _loss_combine_body(ps_ref, out_ref):
    m = jnp.sum(ps_ref[...]) * (1.0 / 4194304.0)
    out_ref[...] = (0.25 * m + m).reshape(1, 1)


def kernel(inputs, embedding_weight):
    B, C, H, W = inputs.shape
    flat_x = jnp.transpose(inputs, (0, 2, 3, 1)).reshape(_N, _D)
    x_b = (2.0 * flat_x).astype(jnp.bfloat16)
    w_b = embedding_weight.astype(jnp.bfloat16)
    x_iso = jax.lax.optimization_barrier(inputs)
    sx = jnp.sum(jnp.transpose(x_iso, (0, 2, 3, 1)).reshape(_N, _D) ** 2,
                 axis=1, keepdims=True)
    sw = jnp.sum(embedding_weight ** 2, axis=1).reshape(1, _K)

    idx2d = pl.pallas_call(
        _dist_argmin_body,
        grid=(_N // _TN,),
        in_specs=[
            pl.BlockSpec((_TN, _D), lambda i: (i, 0)),
            pl.BlockSpec((_K, _D), lambda i: (0, 0)),
            pl.BlockSpec((_TN, 1), lambda i: (i, 0)),
            pl.BlockSpec((1, _K), lambda i: (0, 0)),
        ],
        out_specs=pl.BlockSpec((_TN, 1), lambda i: (i, 0)),
        out_shape=jax.ShapeDtypeStruct((_N, 1), jnp.int32),
    )(x_b, w_b, sx, sw)
    idx_flat = idx2d.reshape(_N)

    q_flat = _sc_gather(embedding_weight, idx_flat)

    st_flat, ps = pl.pallas_call(
        _st_loss_body,
        grid=(16,),
        in_specs=[
            pl.BlockSpec((_N // 16, _D), lambda i: (i, 0)),
            pl.BlockSpec((_N // 16, _D), lambda i: (i, 0)),
        ],
        out_specs=[
            pl.BlockSpec((_N // 16, _D), lambda i: (i, 0)),
            pl.BlockSpec((1, 1, 1), lambda i: (i, 0, 0)),
        ],
        out_shape=[
            jax.ShapeDtypeStruct((_N, _D), jnp.float32),
            jax.ShapeDtypeStruct((16, 1, 1), jnp.float32),
        ],
    )(flat_x, q_flat)

    loss = pl.pallas_call(
        _loss_combine_body,
        in_specs=[pl.BlockSpec((16, 1, 1), lambda: (0, 0, 0))],
        out_specs=pl.BlockSpec((1, 1), lambda: (0, 0)),
        out_shape=jax.ShapeDtypeStruct((1, 1), jnp.float32),
    )(ps)[0, 0]

    quantized_st = jnp.transpose(st_flat.reshape(B, H, W, C), (0, 3, 1, 2))
    return quantized_st, loss, idx_flat.reshape(B, H, W)
